# baseline (speedup 1.0000x reference)
import jax
import jax.numpy as jnp
from jax import lax
from jax.experimental import pallas as pl
from jax.experimental.pallas import tpu as pltpu
from jax.experimental.pallas import tpu_sc as plsc

_W = 256
_NW = 32
_PITCH = 257


def kernel(x, table):
    B, L = x.shape
    V, D = table.shape
    N = B * L
    nbb = B // _W
    bb_per_w = nbb // _NW

    idx = x.T.reshape(N).astype(jnp.int32)
    ih = idx >> 1
    po = (idx & 1) * D
    table2 = table.reshape(V // 2, 2 * D)

    mesh = plsc.VectorSubcoreMesh(core_axis_name="core",
                                  subcore_axis_name="subcore")

    @pl.kernel(
        out_type=jax.ShapeDtypeStruct((L, D, B), table.dtype),
        mesh=mesh,
        compiler_params=pltpu.CompilerParams(use_tc_tiling_on_sc=True,
                                             needs_layout_passes=False),
        scratch_types=[
            pltpu.VMEM((_W,), jnp.int32),
            pltpu.VMEM((_W,), jnp.int32),
            pltpu.VMEM((_W,), jnp.int32),
            pltpu.VMEM((_W,), jnp.int32),
            pltpu.VMEM((_W, 128), jnp.float32),
            pltpu.VMEM((_W, 128), jnp.float32),
            pltpu.VMEM((D, _PITCH), jnp.float32),
            pltpu.SemaphoreType.DMA,
            pltpu.SemaphoreType.DMA,
        ],
    )
    def gather_kernel(table_hbm, ih_hbm, po_hbm, out_hbm, ib0, ib1, ob0, ob1,
                      g0, g1, tb, sem0, sem1):
        wid = lax.axis_index("subcore") * 2 + lax.axis_index("core")
        bb0 = wid * bb_per_w

        row_ids = [jnp.arange(16, dtype=jnp.int32) + 16 * c for c in range(4)]

        def stage_and_fire(l, j, ib, ob, g, sem):
            n0 = l * B + (bb0 + j) * _W
            pltpu.sync_copy(ih_hbm.at[pl.ds(n0, _W)], ib)
            pltpu.sync_copy(po_hbm.at[pl.ds(n0, _W)], ob)
            pltpu.make_async_copy(table_hbm.at[ib], g, sem).start()

        def transpose_and_store(l, j, g, ob):
            @plsc.parallel_loop(0, _W, step=16, unroll=1)
            def _(b0):
                offv = ob[pl.ds(b0, 16)]
                for i in range(16):
                    b = b0 + i
                    off = offv[i]
                    colv = jnp.full((16,), 0, jnp.int32) + b
                    for c in range(4):
                        vals = g[b, pl.ds(off + 16 * c, 16)]
                        plsc.store_scatter(tb, [row_ids[c], colv], vals)
            pltpu.sync_copy(
                tb.at[:, pl.ds(0, _W)],
                out_hbm.at[l, :, pl.ds((bb0 + j) * _W, _W)],
            )

        stage_and_fire(0, 0, ib0, ob0, g0, sem0)

        @pl.loop(0, L)
        def _(l):
            stage_and_fire(l, 1, ib1, ob1, g1, sem1)
            pltpu.make_async_copy(table_hbm.at[ib0], g0, sem0).wait()
            transpose_and_store(l, 0, g0, ob0)
            @pl.when(l < L - 1)
            def _():
                stage_and_fire(l + 1, 0, ib0, ob0, g0, sem0)
            pltpu.make_async_copy(table_hbm.at[ib1], g1, sem1).wait()
            transpose_and_store(l, 1, g1, ob1)

    out = gather_kernel(table2, ih, po)
    return out.transpose(2, 0, 1)


# speedup vs baseline: 1.3277x; 1.3277x over previous
import jax
import jax.numpy as jnp
from jax import lax
from jax.experimental import pallas as pl
from jax.experimental.pallas import tpu as pltpu
from jax.experimental.pallas import tpu_sc as plsc

_W = 256
_NW = 32
_PITCH = 257


def kernel(x, table):
    B, L = x.shape
    V, D = table.shape
    N = B * L
    nbb = B // _W
    bb_per_w = nbb // _NW

    idx = x.T.reshape(N).astype(jnp.int32)

    mesh = plsc.VectorSubcoreMesh(core_axis_name="core",
                                  subcore_axis_name="subcore")

    @pl.kernel(
        out_type=jax.ShapeDtypeStruct((L, B // 128, D // 8, 8, 128),
                                      table.dtype),
        mesh=mesh,
        compiler_params=pltpu.CompilerParams(use_tc_tiling_on_sc=False,
                                             needs_layout_passes=False),
        scratch_types=[
            pltpu.VMEM((_W,), jnp.int32),
            pltpu.VMEM((_W,), jnp.int32),
            pltpu.VMEM((_W, D), jnp.float32),
            pltpu.VMEM((_W, D), jnp.float32),
            pltpu.VMEM((D // 8, 8, _PITCH), jnp.float32),
            pltpu.SemaphoreType.DMA,
            pltpu.SemaphoreType.DMA,
        ],
    )
    def gather_kernel(table_hbm, idx_hbm, out_hbm, ib0, ib1, g0, g1, tb,
                      sem0, sem1):
        wid = lax.axis_index("subcore") * 2 + lax.axis_index("core")
        bb0 = wid * bb_per_w

        lane = jnp.arange(16, dtype=jnp.int32)
        dt_ids = [(16 * c + lane) >> 3 for c in range(4)]
        dr_ids = [(16 * c + lane) & 7 for c in range(4)]

        def stage_and_fire(l, j, ib, g, sem):
            n0 = l * B + (bb0 + j) * _W
            pltpu.sync_copy(idx_hbm.at[pl.ds(n0, _W)], ib)
            pltpu.make_async_copy(table_hbm.at[ib], g, sem).start()

        def transpose_and_store(l, j, g):
            @plsc.parallel_loop(0, _W, step=1, unroll=8)
            def _(b):
                colv = jnp.full((16,), 0, jnp.int32) + b
                for c in range(4):
                    vals = g[b, pl.ds(16 * c, 16)]
                    plsc.store_scatter(tb, [dt_ids[c], dr_ids[c], colv], vals)
            for k in range(2):
                pltpu.sync_copy(
                    tb.at[:, :, pl.ds(128 * k, 128)],
                    out_hbm.at[l, (bb0 + j) * 2 + k],
                )

        stage_and_fire(0, 0, ib0, g0, sem0)

        @pl.loop(0, L)
        def _(l):
            stage_and_fire(l, 1, ib1, g1, sem1)
            pltpu.make_async_copy(table_hbm.at[ib0], g0, sem0).wait()
            transpose_and_store(l, 0, g0)
            @pl.when(l < L - 1)
            def _():
                stage_and_fire(l + 1, 0, ib0, g0, sem0)
            pltpu.make_async_copy(table_hbm.at[ib1], g1, sem1).wait()
            transpose_and_store(l, 1, g1)

    out = gather_kernel(table, idx)
    return out.transpose(1, 4, 0, 2, 3).reshape(B, L, D)


# speedup vs baseline: 1.7645x; 1.3290x over previous
import jax
import jax.numpy as jnp
from jax import lax
from jax.experimental import pallas as pl
from jax.experimental.pallas import tpu as pltpu
from jax.experimental.pallas import tpu_sc as plsc

_W = 256
_NW = 32
_PITCH = 257


def kernel(x, table):
    B, L = x.shape
    V, D = table.shape
    N = B * L
    nbb = B // _W
    bb_per_w = nbb // _NW

    idx = x.T.reshape(N).astype(jnp.int32)

    mesh = plsc.VectorSubcoreMesh(core_axis_name="core",
                                  subcore_axis_name="subcore")

    @pl.kernel(
        out_type=jax.ShapeDtypeStruct((L, D // 8, B // 128, 8, 128),
                                      table.dtype),
        mesh=mesh,
        compiler_params=pltpu.CompilerParams(use_tc_tiling_on_sc=False,
                                             needs_layout_passes=False),
        scratch_types=[
            pltpu.VMEM((_W,), jnp.int32),
            pltpu.VMEM((_W,), jnp.int32),
            pltpu.VMEM((_W, D), jnp.float32),
            pltpu.VMEM((_W, D), jnp.float32),
            pltpu.VMEM((D // 8, 8, _PITCH), jnp.float32),
            pltpu.SemaphoreType.DMA,
            pltpu.SemaphoreType.DMA,
        ],
    )
    def gather_kernel(table_hbm, idx_hbm, out_hbm, ib0, ib1, g0, g1, tb,
                      sem0, sem1):
        wid = lax.axis_index("subcore") * 2 + lax.axis_index("core")
        bb0 = wid * bb_per_w

        lane = jnp.arange(16, dtype=jnp.int32)
        dt_ids = [(16 * c + lane) >> 3 for c in range(4)]
        dr_ids = [(16 * c + lane) & 7 for c in range(4)]

        def stage_and_fire(l, j, ib, g, sem):
            n0 = l * B + (bb0 + j) * _W
            pltpu.sync_copy(idx_hbm.at[pl.ds(n0, _W)], ib)
            pltpu.make_async_copy(table_hbm.at[ib], g, sem).start()

        def transpose_and_store(l, j, g):
            @plsc.parallel_loop(0, _W, step=1, unroll=8)
            def _(b):
                colv = jnp.full((16,), 0, jnp.int32) + b
                for c in range(4):
                    vals = g[b, pl.ds(16 * c, 16)]
                    plsc.store_scatter(tb, [dt_ids[c], dr_ids[c], colv], vals)
            for k in range(2):
                for dt in range(D // 8):
                    pltpu.sync_copy(
                        tb.at[dt, :, pl.ds(128 * k, 128)],
                        out_hbm.at[l, dt, (bb0 + j) * 2 + k],
                    )

        stage_and_fire(0, 0, ib0, g0, sem0)

        @pl.loop(0, L)
        def _(l):
            stage_and_fire(l, 1, ib1, g1, sem1)
            pltpu.make_async_copy(table_hbm.at[ib0], g0, sem0).wait()
            transpose_and_store(l, 0, g0)
            @pl.when(l < L - 1)
            def _():
                stage_and_fire(l + 1, 0, ib0, g0, sem0)
            pltpu.make_async_copy(table_hbm.at[ib1], g1, sem1).wait()
            transpose_and_store(l, 1, g1)

    out = gather_kernel(table, idx)
    return out.transpose(2, 4, 0, 1, 3).reshape(B, L, D)
